# baseline (speedup 1.0000x reference)
import functools
import jax
import jax.numpy as jnp
from jax import lax
from jax.experimental import pallas as pl
from jax.experimental.pallas import tpu as pltpu
from jax.experimental.pallas import tpu_sc as plsc

_N = 12
_K = 4096
_RT = 64
_RP = 128
_TILE = 128
_CH = 16
_NF = 6
_NB = 5
_NPAIR = _N * _N
_NMAP = _RT * _RP
_TAPS = tuple((d, d) for d in range(_NF)) + tuple(
    (_K - _NB + i, _NF + i) for i in range(_NB))
_NW = 32


def _sc_gather(nb, nf):
    bsz = nb * nf
    mesh = plsc.VectorSubcoreMesh(core_axis_name="c", subcore_axis_name="s")
    nloop = (bsz + _NW - 1) // _NW

    @functools.partial(
        pl.kernel, mesh=mesh,
        out_type=[
            jax.ShapeDtypeStruct((bsz, _N, _N, _CH), jnp.float32),
            jax.ShapeDtypeStruct((bsz, _N, _N, _CH), jnp.float32),
        ],
        scratch_types=[
            pltpu.VMEM((_N, _N, _TILE), jnp.float32),
            pltpu.VMEM((_N, _N, _TILE), jnp.float32),
            pltpu.VMEM((_N, _N, _CH), jnp.float32),
            pltpu.VMEM((_N, _N, _CH), jnp.float32),
        ],
    )
    def gather(x_hbm, outf, outb, buf_f, buf_b, cf, cb):
        w = lax.axis_index("s") * 2 + lax.axis_index("c")
        for j in range(nloop):
            bf = w + j * _NW

            @pl.when(bf < bsz)
            def _():
                b = bf // nf
                f = bf % nf
                pltpu.sync_copy(x_hbm.at[b, f, :, :, pl.ds(0, _TILE)],
                                buf_f)
                pltpu.sync_copy(x_hbm.at[b, f, :, :, pl.ds(_K - _TILE,
                                                           _TILE)], buf_b)
                for n in range(_N):
                    for m in range(_N):
                        cf[n, m] = buf_f[n, m, pl.ds(0, _CH)]
                        cb[n, m] = buf_b[n, m, pl.ds(_TILE - _CH, _CH)]
                pltpu.sync_copy(cf, outf.at[bf])
                pltpu.sync_copy(cb, outb.at[bf])

    return gather


def _tc_body(xf_ref, xb_ref, tau_ref, out_ref):
    bsz = xf_ref.shape[0]
    xf = xf_ref[...].reshape(bsz, _NPAIR, _CH)
    xb = xb_ref[...].reshape(bsz, _NPAIR, _CH)
    tau = tau_ref[...]
    acc = None
    for kval, col in _TAPS:
        if col < _NF:
            xsd = xf[:, :, col]
        else:
            xsd = xb[:, :, col - _NF + _CH - _NB]
        mask = (tau == kval).astype(jnp.bfloat16)
        term = jax.lax.dot(xsd.astype(jnp.bfloat16), mask,
                           preferred_element_type=jnp.float32)
        acc = term if acc is None else acc + term
    m = jnp.mean(acc, axis=-1, keepdims=True)
    acc = acc - m + 1e-12
    mx = jnp.max(acc, axis=-1, keepdims=True)
    out_ref[...] = acc / mx


def kernel(x, tau0):
    batch = x.shape[:-3]
    bsz = 1
    for s in batch:
        bsz *= s
    nf = batch[-1] if len(batch) > 1 else bsz
    nb = bsz // nf
    tau_r = tau0.reshape(_NPAIR, _NMAP)
    x5 = x.reshape((nb, nf) + x.shape[-3:])

    xf, xb = _sc_gather(nb, nf)(x5)

    out = pl.pallas_call(
        _tc_body,
        out_shape=jax.ShapeDtypeStruct((bsz, _NMAP), jnp.float32),
        in_specs=[
            pl.BlockSpec(memory_space=pltpu.VMEM),
            pl.BlockSpec(memory_space=pltpu.VMEM),
            pl.BlockSpec(memory_space=pltpu.VMEM),
        ],
        out_specs=pl.BlockSpec(memory_space=pltpu.VMEM),
    )(xf, xb, tau_r)
    return out.reshape(batch + (_RT, _RP))


# speedup vs baseline: 1.0078x; 1.0078x over previous
import functools
import jax
import jax.numpy as jnp
from jax import lax
from jax.experimental import pallas as pl
from jax.experimental.pallas import tpu as pltpu
from jax.experimental.pallas import tpu_sc as plsc

_N = 12
_K = 4096
_RT = 64
_RP = 128
_TILE = 128
_CH = 16
_NF = 6
_NB = 5
_NPAIR = _N * _N
_NMAP = _RT * _RP
_TAPS = tuple((d, d) for d in range(_NF)) + tuple(
    (_K - _NB + i, _NF + i) for i in range(_NB))
_NW = 32


def _sc_gather(nb, nf):
    bsz = nb * nf
    mesh = plsc.VectorSubcoreMesh(core_axis_name="c", subcore_axis_name="s")
    nloop = (bsz + _NW - 1) // _NW

    @functools.partial(
        pl.kernel, mesh=mesh,
        out_type=[
            jax.ShapeDtypeStruct((bsz, _N, _N, _CH), jnp.float32),
            jax.ShapeDtypeStruct((bsz, _N, _N, _CH), jnp.float32),
        ],
        scratch_types=[
            pltpu.VMEM((_N, _N, _TILE), jnp.float32),
            pltpu.VMEM((_N, _N, _TILE), jnp.float32),
            pltpu.VMEM((_N, _N, _CH), jnp.float32),
            pltpu.VMEM((_N, _N, _CH), jnp.float32),
            pltpu.SemaphoreType.DMA,
            pltpu.SemaphoreType.DMA,
            pltpu.SemaphoreType.DMA,
            pltpu.SemaphoreType.DMA,
        ],
    )
    def gather(x_hbm, outf, outb, buf_f, buf_b, cf, cb, s0, s1, s2, s3):
        w = lax.axis_index("s") * 2 + lax.axis_index("c")
        for j in range(nloop):
            bf = w + j * _NW

            @pl.when(bf < bsz)
            def _():
                b = bf // nf
                f = bf % nf
                df = pltpu.make_async_copy(
                    x_hbm.at[b, f, :, :, pl.ds(0, _TILE)], buf_f, s0)
                db = pltpu.make_async_copy(
                    x_hbm.at[b, f, :, :, pl.ds(_K - _TILE, _TILE)],
                    buf_b, s1)
                df.start()
                db.start()
                df.wait()
                db.wait()
                for n in range(_N):
                    for m in range(_N):
                        cf[n, m] = buf_f[n, m, pl.ds(0, _CH)]
                        cb[n, m] = buf_b[n, m, pl.ds(_TILE - _CH, _CH)]
                of = pltpu.make_async_copy(cf, outf.at[bf], s2)
                ob = pltpu.make_async_copy(cb, outb.at[bf], s3)
                of.start()
                ob.start()
                of.wait()
                ob.wait()

    return gather


def _tc_body(xf_ref, xb_ref, tau_ref, out_ref):
    bsz = xf_ref.shape[0]
    xf = xf_ref[...].reshape(bsz, _NPAIR, _CH)
    xb = xb_ref[...].reshape(bsz, _NPAIR, _CH)
    tau = tau_ref[...]
    acc = None
    for kval, col in _TAPS:
        if col < _NF:
            xsd = xf[:, :, col]
        else:
            xsd = xb[:, :, col - _NF + _CH - _NB]
        mask = (tau == kval).astype(jnp.bfloat16)
        term = jax.lax.dot(xsd.astype(jnp.bfloat16), mask,
                           preferred_element_type=jnp.float32)
        acc = term if acc is None else acc + term
    m = jnp.mean(acc, axis=-1, keepdims=True)
    acc = acc - m + 1e-12
    mx = jnp.max(acc, axis=-1, keepdims=True)
    out_ref[...] = acc / mx


def kernel(x, tau0):
    batch = x.shape[:-3]
    bsz = 1
    for s in batch:
        bsz *= s
    nf = batch[-1] if len(batch) > 1 else bsz
    nb = bsz // nf
    tau_r = tau0.reshape(_NPAIR, _NMAP)
    x5 = x.reshape((nb, nf) + x.shape[-3:])

    xf, xb = _sc_gather(nb, nf)(x5)

    out = pl.pallas_call(
        _tc_body,
        out_shape=jax.ShapeDtypeStruct((bsz, _NMAP), jnp.float32),
        in_specs=[
            pl.BlockSpec(memory_space=pltpu.VMEM),
            pl.BlockSpec(memory_space=pltpu.VMEM),
            pl.BlockSpec(memory_space=pltpu.VMEM),
        ],
        out_specs=pl.BlockSpec(memory_space=pltpu.VMEM),
    )(xf, xb, tau_r)
    return out.reshape(batch + (_RT, _RP))
